# baseline (speedup 1.0000x reference)
import functools

import jax
import jax.numpy as jnp
from jax import lax
from jax.experimental import pallas as pl
from jax.experimental.pallas import tpu as pltpu
import jax.experimental.pallas.tpu_sc as plsc

N = 10000
E = 320000
D_NODE = 24
H = 128

NC, NS, L = 2, 16, 16
NW = NC * NS
E_PAD = 327680
EPW = E_PAD // NW
CH_G = 640
CPW_G = EPW // CH_G
CH_S = 128
CPW_S = EPW // CH_S
ROWS_PER_TILE = 632
ACC_ROWS = NS * ROWS_PER_TILE
PACK_ROWS = E_PAD // 8
BLK_E = 1024
NBLK_E = PACK_ROWS // BLK_E

_MESH = plsc.VectorSubcoreMesh(
    core_axis_name="c", subcore_axis_name="s", num_cores=NC, num_subcores=NS
)
_SC_PARAMS = pltpu.CompilerParams(use_tc_tiling_on_sc=False)


@functools.partial(
    pl.kernel,
    out_type=[
        jax.ShapeDtypeStruct((PACK_ROWS, 128), jnp.float32),
        jax.ShapeDtypeStruct((PACK_ROWS, 128), jnp.float32),
    ],
    mesh=_MESH,
    compiler_params=_SC_PARAMS,
    scratch_types=[
        pltpu.VMEM((EPW,), jnp.int32),
        pltpu.VMEM((EPW,), jnp.int32),
        pltpu.VMEM((CH_G, 16), jnp.float32),
        pltpu.VMEM((CH_G, 16), jnp.float32),
        pltpu.VMEM((CH_G, 16), jnp.float32),
        pltpu.VMEM((CH_G, 16), jnp.float32),
        pltpu.SemaphoreType.DMA,
        pltpu.SemaphoreType.DMA,
        pltpu.SemaphoreType.DMA,
        pltpu.SemaphoreType.DMA,
    ],
)
def _gather_kernel(table, srcp, dstp, sout, dout,
                   sidx, didx, sb0, sb1, db0, db1, g0, g1, w0, w1):
    c = lax.axis_index("c")
    s = lax.axis_index("s")
    wid = s * NC + c
    base = wid * EPW
    pltpu.sync_copy(srcp.at[pl.ds(base, EPW)], sidx)
    pltpu.sync_copy(dstp.at[pl.ds(base, EPW)], didx)

    j = wid // 4
    rbase = (wid % 4) * EPW

    sbufs, dbufs = [sb0, sb1], [db0, db1]
    gsems, wsems = [g0, g1], [w0, w1]

    def issue(i, b):
        return [
            pltpu.async_copy(table.at[sidx.at[pl.ds(i * CH_G, CH_G)]],
                             sbufs[b], gsems[b]),
            pltpu.async_copy(table.at[didx.at[pl.ds(i * CH_G, CH_G)]],
                             dbufs[b], gsems[b]),
        ]

    def write(i, b):
        r0 = rbase + i * CH_G
        return [
            pltpu.async_copy(
                sbufs[b], sout.at[pl.ds(r0, CH_G), pl.ds(16 * j, 16)],
                wsems[b]),
            pltpu.async_copy(
                dbufs[b], dout.at[pl.ds(r0, CH_G), pl.ds(16 * j, 16)],
                wsems[b]),
        ]

    pend_g = [None, None]
    pend_w = [None, None]
    pend_g[0] = issue(0, 0)
    for i in range(CPW_G):
        b = i % 2
        if i + 1 < CPW_G:
            if pend_w[1 - b] is not None:
                for h in pend_w[1 - b]:
                    h.wait()
            pend_g[1 - b] = issue(i + 1, 1 - b)
        for h in pend_g[b]:
            h.wait()
        pend_w[b] = write(i, b)
    for p in pend_w:
        if p is not None:
            for h in p:
                h.wait()


def _featurize(s, r):
    d = s - r
    d2 = d * d
    t = d2 + jnp.roll(d2, -1, axis=1) + jnp.roll(d2, -2, axis=1)
    nm = jnp.sqrt(t)
    lane = jax.lax.broadcasted_iota(jnp.int32, d.shape, 1) % 16
    return jnp.where(
        lane < 3, d,
        jnp.where(
            lane == 3, jnp.roll(nm, 3, axis=1),
            jnp.where(lane < 7, jnp.roll(d, 1, axis=1),
                      jnp.roll(nm, 4, axis=1)),
        ),
    )


def _stats_body(sref, rref, oref):
    F = _featurize(sref[...], rref[...])
    s1 = jnp.sum(F, axis=0, keepdims=True)
    s2 = jnp.sum(F * F, axis=0, keepdims=True)

    @pl.when(pl.program_id(0) == 0)
    def _():
        oref[...] = jnp.zeros_like(oref)

    oref[0:1, :] += s1
    oref[1:2, :] += s2


def _stats_kernel(srows, drows):
    return pl.pallas_call(
        _stats_body,
        grid=(NBLK_E,),
        in_specs=[
            pl.BlockSpec((BLK_E, 128), lambda i: (i, 0)),
            pl.BlockSpec((BLK_E, 128), lambda i: (i, 0)),
        ],
        out_specs=pl.BlockSpec((2, 128), lambda i: (0, 0)),
        out_shape=jax.ShapeDtypeStruct((2, 128), jnp.float32),
    )(srows, drows)


def _edge_mlp_body(sref, rref, wref, bref, oref):
    F = _featurize(sref[...], rref[...])
    for j in range(8):
        ef = lax.slice(F, (0, 16 * j), (F.shape[0], 16 * j + 8))
        h = lax.dot_general(
            ef, wref[...], (((1,), (0,)), ((), ())),
            preferred_element_type=jnp.float32,
            precision=lax.Precision.HIGHEST,
        )
        oref[j] = jnp.maximum(h + bref[...], 0.0)


def _edge_mlp(srows, drows, Wp, bp):
    return pl.pallas_call(
        _edge_mlp_body,
        grid=(NBLK_E,),
        in_specs=[
            pl.BlockSpec((BLK_E, 128), lambda i: (i, 0)),
            pl.BlockSpec((BLK_E, 128), lambda i: (i, 0)),
            pl.BlockSpec((8, H), lambda i: (0, 0)),
            pl.BlockSpec((1, H), lambda i: (0, 0)),
        ],
        out_specs=pl.BlockSpec((8, BLK_E, H), lambda i: (0, i, 0)),
        out_shape=jax.ShapeDtypeStruct((8, PACK_ROWS, H), jnp.float32),
    )(srows, drows, Wp, bp)


@functools.partial(
    pl.kernel,
    out_type=jax.ShapeDtypeStruct((NC, ACC_ROWS, H), jnp.float32),
    mesh=_MESH,
    scratch_types=[
        pltpu.VMEM((EPW,), jnp.int32),
        pltpu.VMEM((CH_S, H), jnp.float32),
        pltpu.VMEM((CH_S, H), jnp.float32),
        pltpu.VMEM_SHARED((ACC_ROWS, H), jnp.float32),
        pltpu.SemaphoreType.DMA,
        pltpu.SemaphoreType.DMA,
    ],
)
def _scatter_kernel(he, dstp, zeros_hbm, out_hbm,
                    didx, rb0, rb1, acc_sh, l0, l1):
    c = lax.axis_index("c")
    s = lax.axis_index("s")
    wid = s * NC + c
    base = wid * EPW

    pltpu.sync_copy(
        zeros_hbm.at[pl.ds(s * ROWS_PER_TILE, ROWS_PER_TILE)],
        acc_sh.at[pl.ds(s * ROWS_PER_TILE, ROWS_PER_TILE)],
    )
    pltpu.sync_copy(dstp.at[pl.ds(base, EPW)], didx)
    plsc.subcore_barrier()

    bufs, sems = [rb0, rb1], [l0, l1]

    def issue(i, b):
        return pltpu.async_copy(
            he.at[pl.ds(base + i * CH_S, CH_S)], bufs[b], sems[b]
        )

    pend = [None, None]
    pend[0] = issue(0, 0)
    for i in range(CPW_S):
        b = i % 2
        if i + 1 < CPW_S:
            pend[1 - b] = issue(i + 1, 1 - b)
        pend[b].wait()
        pltpu.sync_copy(
            bufs[b], acc_sh.at[didx.at[pl.ds(i * CH_S, CH_S)]], add=True
        )

    plsc.subcore_barrier()
    pltpu.sync_copy(
        acc_sh.at[pl.ds(s * ROWS_PER_TILE, ROWS_PER_TILE)],
        out_hbm.at[c].at[pl.ds(s * ROWS_PER_TILE, ROWS_PER_TILE)],
    )


def _node_mlp_body(x_ref, a_ref, b_ref, wx_ref, wa_ref, wo_ref, out_ref):
    agg = a_ref[...] + b_ref[...]
    dot = functools.partial(
        lax.dot_general,
        dimension_numbers=(((1,), (0,)), ((), ())),
        preferred_element_type=jnp.float32,
        precision=lax.Precision.HIGHEST,
    )
    h = jnp.maximum(dot(x_ref[...], wx_ref[...]) + dot(agg, wa_ref[...]), 0.0)
    out_ref[...] = dot(h, wo_ref[...])


def _node_mlp(x, aggs, Wn_x, Wn_a, W_out):
    BLK = 2000
    agg_spec = pl.BlockSpec((BLK, H), lambda i: (i, 0))
    return pl.pallas_call(
        _node_mlp_body,
        grid=(N // BLK,),
        in_specs=[
            pl.BlockSpec((BLK, D_NODE), lambda i: (i, 0)),
            agg_spec, agg_spec,
            pl.BlockSpec((D_NODE, H), lambda i: (0, 0)),
            pl.BlockSpec((H, H), lambda i: (0, 0)),
            pl.BlockSpec((H, 3), lambda i: (0, 0)),
        ],
        out_specs=pl.BlockSpec((BLK, 3), lambda i: (i, 0)),
        out_shape=jax.ShapeDtypeStruct((N, 3), jnp.float32),
    )(x, *aggs, Wn_x, Wn_a, W_out)


def kernel(pos, rest_pos, x, edge_index, W_edge, W_node, W_out):
    src = edge_index[0].astype(jnp.int32)
    dst = edge_index[1].astype(jnp.int32)
    table = jnp.concatenate(
        [pos, rest_pos, jnp.zeros((N, 10), jnp.float32)], axis=1
    )
    pad = E_PAD - E
    srcp = jnp.concatenate([src, jnp.zeros((pad,), jnp.int32)])
    dstp0 = jnp.concatenate([dst, jnp.zeros((pad,), jnp.int32)])
    dstpN = jnp.concatenate([dst, jnp.full((pad,), N, jnp.int32)])

    srows, drows = _gather_kernel(table, srcp, dstp0)
    srows = srows.reshape(PACK_ROWS, 128)
    drows = drows.reshape(PACK_ROWS, 128)

    stats = _stats_kernel(srows, drows)
    stats = stats.reshape(2, 8, 16).sum(axis=1)
    mu = stats[0, :8] / E
    var = jnp.maximum(stats[1, :8] / E - mu * mu, 0.0)
    sigma = jnp.sqrt(var) + 1e-8
    Wp = W_edge / sigma[:, None]
    bp = (-(mu / sigma))[None, :] @ W_edge

    he = _edge_mlp(srows, drows, Wp, bp).reshape(E_PAD, H)
    zeros_acc = jnp.zeros((ACC_ROWS, H), jnp.float32)
    parts = _scatter_kernel(he, dstpN, zeros_acc)

    return _node_mlp(x, [parts[0], parts[1]], W_node[:D_NODE], W_node[D_NODE:],
                     W_out)


# speedup vs baseline: 1.3858x; 1.3858x over previous
import functools

import jax
import jax.numpy as jnp
from jax import lax
from jax.experimental import pallas as pl
from jax.experimental.pallas import tpu as pltpu
import jax.experimental.pallas.tpu_sc as plsc

N = 10000
E = 320000
D_NODE = 24
H = 128

NC, NS, L = 2, 16, 16
NW = NC * NS
E_PAD = 327680
E_HALF = E_PAD // 2
EPW = E_HALF // NW
CH_G = 640
CPW_G = EPW // CH_G
CH_S = 128
CPW_S = EPW // CH_S
ROWS_PER_TILE = 632
ACC_ROWS = NS * ROWS_PER_TILE
PACK_ROWS = E_HALF // 8
BLK_E = 1024
NBLK_E = PACK_ROWS // BLK_E

_MESH = plsc.VectorSubcoreMesh(
    core_axis_name="c", subcore_axis_name="s", num_cores=NC, num_subcores=NS
)
_SC_PARAMS = pltpu.CompilerParams(use_tc_tiling_on_sc=False)


@functools.partial(
    pl.kernel,
    out_type=[
        jax.ShapeDtypeStruct((PACK_ROWS, 128), jnp.float32),
        jax.ShapeDtypeStruct((PACK_ROWS, 128), jnp.float32),
    ],
    mesh=_MESH,
    compiler_params=_SC_PARAMS,
    scratch_types=[
        pltpu.VMEM((EPW,), jnp.int32),
        pltpu.VMEM((EPW,), jnp.int32),
        pltpu.VMEM((CH_G, 16), jnp.float32),
        pltpu.VMEM((CH_G, 16), jnp.float32),
        pltpu.VMEM((CH_G, 16), jnp.float32),
        pltpu.VMEM((CH_G, 16), jnp.float32),
        pltpu.VMEM_SHARED((ACC_ROWS, 16), jnp.float32),
        pltpu.SemaphoreType.DMA,
        pltpu.SemaphoreType.DMA,
        pltpu.SemaphoreType.DMA,
        pltpu.SemaphoreType.DMA,
    ],
)
def _gather_kernel(table, srcp, dstp, sout, dout,
                   sidx, didx, sb0, sb1, db0, db1, tab_sh, g0, g1, w0, w1):
    c = lax.axis_index("c")
    s = lax.axis_index("s")
    wid = s * NC + c
    base = wid * EPW
    pltpu.sync_copy(
        table.at[pl.ds(s * ROWS_PER_TILE, ROWS_PER_TILE)],
        tab_sh.at[pl.ds(s * ROWS_PER_TILE, ROWS_PER_TILE)],
    )
    pltpu.sync_copy(srcp.at[pl.ds(base, EPW)], sidx)
    pltpu.sync_copy(dstp.at[pl.ds(base, EPW)], didx)
    plsc.subcore_barrier()

    j = wid // 4
    rbase = (wid % 4) * EPW

    sbufs, dbufs = [sb0, sb1], [db0, db1]
    gsems, wsems = [g0, g1], [w0, w1]

    def issue(i, b):
        return [
            pltpu.async_copy(tab_sh.at[sidx.at[pl.ds(i * CH_G, CH_G)]],
                             sbufs[b], gsems[b]),
            pltpu.async_copy(tab_sh.at[didx.at[pl.ds(i * CH_G, CH_G)]],
                             dbufs[b], gsems[b]),
        ]

    def write(i, b):
        r0 = rbase + i * CH_G
        return [
            pltpu.async_copy(
                sbufs[b], sout.at[pl.ds(r0, CH_G), pl.ds(16 * j, 16)],
                wsems[b]),
            pltpu.async_copy(
                dbufs[b], dout.at[pl.ds(r0, CH_G), pl.ds(16 * j, 16)],
                wsems[b]),
        ]

    pend_g = [None, None]
    pend_w = [None, None]
    pend_g[0] = issue(0, 0)
    for i in range(CPW_G):
        b = i % 2
        if i + 1 < CPW_G:
            if pend_w[1 - b] is not None:
                for h in pend_w[1 - b]:
                    h.wait()
            pend_g[1 - b] = issue(i + 1, 1 - b)
        for h in pend_g[b]:
            h.wait()
        pend_w[b] = write(i, b)
    for p in pend_w:
        if p is not None:
            for h in p:
                h.wait()


def _featurize(s, r):
    d = s - r
    d2 = d * d
    t = d2 + jnp.roll(d2, -1, axis=1) + jnp.roll(d2, -2, axis=1)
    nm = jnp.sqrt(t)
    lane = jax.lax.broadcasted_iota(jnp.int32, d.shape, 1) % 16
    return jnp.where(
        lane < 3, d,
        jnp.where(
            lane == 3, jnp.roll(nm, 3, axis=1),
            jnp.where(lane < 7, jnp.roll(d, 1, axis=1),
                      jnp.roll(nm, 4, axis=1)),
        ),
    )


def _stats_body(sref, rref, oref):
    F = _featurize(sref[...], rref[...])
    s1 = jnp.sum(F, axis=0, keepdims=True)
    s2 = jnp.sum(F * F, axis=0, keepdims=True)

    @pl.when(pl.program_id(0) == 0)
    def _():
        oref[...] = jnp.zeros_like(oref)

    oref[0:1, :] += s1
    oref[1:2, :] += s2


def _stats_kernel(srows, drows):
    return pl.pallas_call(
        _stats_body,
        grid=(NBLK_E,),
        in_specs=[
            pl.BlockSpec((BLK_E, 128), lambda i: (i, 0)),
            pl.BlockSpec((BLK_E, 128), lambda i: (i, 0)),
        ],
        out_specs=pl.BlockSpec((2, 128), lambda i: (0, 0)),
        out_shape=jax.ShapeDtypeStruct((2, 128), jnp.float32),
    )(srows, drows)


def _edge_mlp_body(sref, rref, wref, bref, oref):
    F = _featurize(sref[...], rref[...])
    for j in range(8):
        ef = lax.slice(F, (0, 16 * j), (F.shape[0], 16 * j + 8))
        h = lax.dot_general(
            ef, wref[...], (((1,), (0,)), ((), ())),
            preferred_element_type=jnp.float32,
            precision=lax.Precision.HIGHEST,
        )
        oref[j] = jnp.maximum(h + bref[...], 0.0)


def _edge_mlp(srows, drows, Wp, bp):
    return pl.pallas_call(
        _edge_mlp_body,
        grid=(NBLK_E,),
        in_specs=[
            pl.BlockSpec((BLK_E, 128), lambda i: (i, 0)),
            pl.BlockSpec((BLK_E, 128), lambda i: (i, 0)),
            pl.BlockSpec((8, H), lambda i: (0, 0)),
            pl.BlockSpec((1, H), lambda i: (0, 0)),
        ],
        out_specs=pl.BlockSpec((8, BLK_E, H), lambda i: (0, i, 0)),
        out_shape=jax.ShapeDtypeStruct((8, PACK_ROWS, H), jnp.float32),
    )(srows, drows, Wp, bp)


@functools.partial(
    pl.kernel,
    out_type=jax.ShapeDtypeStruct((NC, ACC_ROWS, H), jnp.float32),
    mesh=_MESH,
    scratch_types=[
        pltpu.VMEM((EPW,), jnp.int32),
        pltpu.VMEM((CH_S, H), jnp.float32),
        pltpu.VMEM((CH_S, H), jnp.float32),
        pltpu.VMEM_SHARED((ACC_ROWS, H), jnp.float32),
        pltpu.SemaphoreType.DMA,
        pltpu.SemaphoreType.DMA,
    ],
)
def _scatter_kernel(he, dstp, zeros_hbm, out_hbm,
                    didx, rb0, rb1, acc_sh, l0, l1):
    c = lax.axis_index("c")
    s = lax.axis_index("s")
    wid = s * NC + c
    base = wid * EPW

    pltpu.sync_copy(
        zeros_hbm.at[pl.ds(s * ROWS_PER_TILE, ROWS_PER_TILE)],
        acc_sh.at[pl.ds(s * ROWS_PER_TILE, ROWS_PER_TILE)],
    )
    pltpu.sync_copy(dstp.at[pl.ds(base, EPW)], didx)
    plsc.subcore_barrier()

    bufs, sems = [rb0, rb1], [l0, l1]

    def issue(i, b):
        return pltpu.async_copy(
            he.at[pl.ds(base + i * CH_S, CH_S)], bufs[b], sems[b]
        )

    pend = [None, None]
    pend[0] = issue(0, 0)
    for i in range(CPW_S):
        b = i % 2
        if i + 1 < CPW_S:
            pend[1 - b] = issue(i + 1, 1 - b)
        pend[b].wait()
        pltpu.sync_copy(
            bufs[b], acc_sh.at[didx.at[pl.ds(i * CH_S, CH_S)]], add=True
        )

    plsc.subcore_barrier()
    pltpu.sync_copy(
        acc_sh.at[pl.ds(s * ROWS_PER_TILE, ROWS_PER_TILE)],
        out_hbm.at[c].at[pl.ds(s * ROWS_PER_TILE, ROWS_PER_TILE)],
    )


def _node_mlp_body(x_ref, a_ref, b_ref, c_ref, d_ref,
                   wx_ref, wa_ref, wo_ref, out_ref):
    agg = (a_ref[...] + b_ref[...]) + (c_ref[...] + d_ref[...])
    dot = functools.partial(
        lax.dot_general,
        dimension_numbers=(((1,), (0,)), ((), ())),
        preferred_element_type=jnp.float32,
        precision=lax.Precision.HIGHEST,
    )
    h = jnp.maximum(dot(x_ref[...], wx_ref[...]) + dot(agg, wa_ref[...]), 0.0)
    out_ref[...] = dot(h, wo_ref[...])


def _node_mlp(x, aggs, Wn_x, Wn_a, W_out):
    BLK = 2000
    agg_spec = pl.BlockSpec((BLK, H), lambda i: (i, 0))
    return pl.pallas_call(
        _node_mlp_body,
        grid=(N // BLK,),
        in_specs=[
            pl.BlockSpec((BLK, D_NODE), lambda i: (i, 0)),
            agg_spec, agg_spec, agg_spec, agg_spec,
            pl.BlockSpec((D_NODE, H), lambda i: (0, 0)),
            pl.BlockSpec((H, H), lambda i: (0, 0)),
            pl.BlockSpec((H, 3), lambda i: (0, 0)),
        ],
        out_specs=pl.BlockSpec((BLK, 3), lambda i: (i, 0)),
        out_shape=jax.ShapeDtypeStruct((N, 3), jnp.float32),
    )(x, *aggs, Wn_x, Wn_a, W_out)


def kernel(pos, rest_pos, x, edge_index, W_edge, W_node, W_out):
    src = edge_index[0].astype(jnp.int32)
    dst = edge_index[1].astype(jnp.int32)
    table = jnp.concatenate(
        [pos, rest_pos, jnp.zeros((N, 10), jnp.float32)], axis=1
    )
    table = jnp.concatenate(
        [table, jnp.zeros((ACC_ROWS - N, 16), jnp.float32)], axis=0
    )
    pad = E_PAD - E
    srcp = jnp.concatenate([src, jnp.zeros((pad,), jnp.int32)])
    dstp0 = jnp.concatenate([dst, jnp.zeros((pad,), jnp.int32)])
    dstpN = jnp.concatenate([dst, jnp.full((pad,), N, jnp.int32)])

    halves = []
    for hidx in range(2):
        sl = slice(hidx * E_HALF, (hidx + 1) * E_HALF)
        srows, drows = _gather_kernel(table, srcp[sl], dstp0[sl])
        halves.append((srows.reshape(PACK_ROWS, 128),
                       drows.reshape(PACK_ROWS, 128),
                       dstpN[sl]))

    stats = sum(_stats_kernel(s, r) for s, r, _ in halves)
    stats = stats.reshape(2, 8, 16).sum(axis=1)
    mu = stats[0, :8] / E
    var = jnp.maximum(stats[1, :8] / E - mu * mu, 0.0)
    sigma = jnp.sqrt(var) + 1e-8
    Wp = W_edge / sigma[:, None]
    bp = (-(mu / sigma))[None, :] @ W_edge

    zeros_acc = jnp.zeros((ACC_ROWS, H), jnp.float32)
    aggs = []
    for srows, drows, dst_h in halves:
        he = _edge_mlp(srows, drows, Wp, bp).reshape(E_HALF, H)
        parts = _scatter_kernel(he, dst_h, zeros_acc)
        aggs.extend([parts[0], parts[1]])

    return _node_mlp(x, aggs, W_node[:D_NODE], W_node[D_NODE:], W_out)
